# baseline (device time: 394719 ns/iter reference)
import jax
import jax.numpy as jnp
from jax import lax
from jax.experimental import pallas as pl
from jax.experimental.pallas import tpu as pltpu

N_DEV = 16
M = 4096
N = 2048
M_CH = M // N_DEV
N_HALF = N // 2
N_SUB = N_HALF // 2
N_SLOT = 4
N_HOPS = 2 * (N_DEV - 1)
RS_HOPS = N_DEV - 1


def kernel(x, w_mat):
    def body(x_ref, w_ref, out_ref,
             comm_r, comm_l, send_sems_r, recv_sems_r, send_sems_l,
             recv_sems_l, credit_r, credit_l,
             bt_send, bt_recv, bt_send_sems, bt_recv_sems):
        d = lax.axis_index("i")
        right = jnp.mod(d + 1, N_DEV)
        left = jnp.mod(d + N_DEV - 1, N_DEV)

        def rows(c):
            return pl.ds(c * M_CH, M_CH)

        def cols(ring, s):
            return pl.ds(ring * N_HALF + s * N_SUB, N_SUB)

        def gemm_half(c, ring):
            out_ref[rows(c), pl.ds(ring * N_HALF, N_HALF)] = jnp.dot(
                x_ref[rows(c), :],
                w_ref[:, pl.ds(ring * N_HALF, N_HALF)],
                preferred_element_type=jnp.float32,
                precision=lax.Precision.HIGHEST,
            )

        def desc(ring, s, slot, rslot, nbr):
            comm = (comm_r, comm_l)[ring]
            ssem = (send_sems_r, send_sems_l)[ring]
            rsem = (recv_sems_r, recv_sems_l)[ring]
            return pltpu.make_async_remote_copy(
                src_ref=comm.at[slot, s],
                dst_ref=comm.at[rslot, s],
                send_sem=ssem.at[slot, s],
                recv_sem=rsem.at[rslot, s],
                device_id=(nbr,),
                device_id_type=pl.DeviceIdType.MESH,
            )

        def send_desc(ring, s, g):
            nbr = right if ring == 0 else left
            return desc(ring, s, jnp.mod(g, N_SLOT), jnp.mod(g + 1, N_SLOT), nbr)

        def retire_and_credit(h):
            @pl.when(h >= 1)
            def _():
                for ring in range(2):
                    for s in range(2):
                        send_desc(ring, s, h - 1).wait_send()

            @pl.when(jnp.logical_and(h >= 2, h <= N_HOPS - 3))
            def _():
                for s in range(2):
                    pl.semaphore_signal(
                        credit_r.at[s], inc=1,
                        device_id=(left,), device_id_type=pl.DeviceIdType.MESH,
                    )
                    pl.semaphore_signal(
                        credit_l.at[s], inc=1,
                        device_id=(right,), device_id_type=pl.DeviceIdType.MESH,
                    )

        def recv_fwd(ring, s, h, do_add, c_rs):
            comm = (comm_r, comm_l)[ring]
            credit = (credit_r, credit_l)[ring]
            rslot = jnp.mod(h + 1, N_SLOT)
            send_desc(ring, s, h).wait_recv()
            if do_add:
                comm[rslot, s] = (
                    comm[rslot, s] + out_ref[rows(c_rs), cols(ring, s)]
                )

            @pl.when(h < N_HOPS - 1)
            def _():
                @pl.when(h + 1 >= N_SLOT)
                def _():
                    pl.semaphore_wait(credit.at[s], 1)
                send_desc(ring, s, h + 1).start()

        for s in range(2):
            comm_r[0, s] = jnp.dot(
                x_ref[rows(d), :], w_ref[:, pl.ds(s * N_SUB, N_SUB)],
                preferred_element_type=jnp.float32,
                precision=lax.Precision.HIGHEST,
            )
            comm_l[0, s] = jnp.dot(
                x_ref[rows(d), :], w_ref[:, pl.ds(N_HALF + s * N_SUB, N_SUB)],
                preferred_element_type=jnp.float32,
                precision=lax.Precision.HIGHEST,
            )

        barrier_sem = pltpu.get_barrier_semaphore()
        for nbr in (left, right):
            pl.semaphore_signal(
                barrier_sem, inc=1,
                device_id=(nbr,), device_id_type=pl.DeviceIdType.MESH,
            )
        pl.semaphore_wait(barrier_sem, 2)

        for ring in range(2):
            for s in range(2):
                send_desc(ring, s, 0).start()
        gemm_half(left, 0)
        gemm_half(right, 1)

        def rs_hop(h, carry):
            retire_and_credit(h)
            c_r = jnp.mod(d - 1 - h + 4 * N_DEV, N_DEV)
            c_l = jnp.mod(d + 1 + h, N_DEV)
            for s in range(2):
                recv_fwd(0, s, h, True, c_r)
                recv_fwd(1, s, h, True, c_l)

            rslot = jnp.mod(h + 1, N_SLOT)

            @pl.when(h == RS_HOPS - 1)
            def _():
                for s in range(2):
                    out_ref[rows(c_r), cols(0, s)] = comm_r[rslot, s]
                    out_ref[rows(c_l), cols(1, s)] = comm_l[rslot, s]

            @pl.when(h <= RS_HOPS - 2)
            def _():
                gemm_half(jnp.mod(d - 2 - h + 4 * N_DEV, N_DEV), 0)
                gemm_half(jnp.mod(d + 2 + h, N_DEV), 1)

            return carry

        lax.fori_loop(0, RS_HOPS, rs_hop, jnp.int32(0))

        own_r = jnp.mod(d + 1, N_DEV)
        own_l = jnp.mod(d + N_DEV - 1, N_DEV)
        val = jnp.maximum(
            jnp.max(jnp.abs(out_ref[rows(own_r), : N_HALF])),
            jnp.max(jnp.abs(out_ref[rows(own_l), N_HALF:])),
        )
        for k in range(4):
            partner = jnp.bitwise_xor(d, 2 ** k)
            bt_send[k] = jnp.full((8, 128), val, jnp.float32)
            bt = pltpu.make_async_remote_copy(
                src_ref=bt_send.at[k],
                dst_ref=bt_recv.at[k],
                send_sem=bt_send_sems.at[k],
                recv_sem=bt_recv_sems.at[k],
                device_id=(partner,),
                device_id_type=pl.DeviceIdType.MESH,
            )
            bt.start()
            bt.wait()
            val = jnp.maximum(val, jnp.max(bt_recv[k]))
        scale = val / 448.0

        def quant(v):
            return (v / scale).astype(jnp.float8_e4m3fn).astype(jnp.float32) * scale

        out_ref[rows(own_r), : N_HALF] = quant(out_ref[rows(own_r), : N_HALF])
        out_ref[rows(own_l), N_HALF:] = quant(out_ref[rows(own_l), N_HALF:])

        def ag_hop(h, carry):
            retire_and_credit(h)
            for s in range(2):
                recv_fwd(0, s, h, False, None)
                recv_fwd(1, s, h, False, None)
            rslot = jnp.mod(h + 1, N_SLOT)
            c_ag_r = jnp.mod(d - h + N_DEV - 1 + 4 * N_DEV, N_DEV)
            c_ag_l = jnp.mod(d + h - N_DEV + 1 + 4 * N_DEV, N_DEV)
            for s in range(2):
                out_ref[rows(c_ag_r), cols(0, s)] = quant(comm_r[rslot, s])
                out_ref[rows(c_ag_l), cols(1, s)] = quant(comm_l[rslot, s])
            return carry

        lax.fori_loop(RS_HOPS, N_HOPS, ag_hop, jnp.int32(0))

        for ring in range(2):
            for s in range(2):
                send_desc(ring, s, N_HOPS - 1).wait_send()

    return pl.pallas_call(
        body,
        out_shape=jax.ShapeDtypeStruct((M, N), jnp.float32),
        in_specs=[
            pl.BlockSpec(memory_space=pltpu.VMEM),
            pl.BlockSpec(memory_space=pltpu.VMEM),
        ],
        out_specs=pl.BlockSpec(memory_space=pltpu.VMEM),
        scratch_shapes=[
            pltpu.VMEM((N_SLOT, 2, M_CH, N_SUB), jnp.float32),
            pltpu.VMEM((N_SLOT, 2, M_CH, N_SUB), jnp.float32),
            pltpu.SemaphoreType.DMA((N_SLOT, 2)),
            pltpu.SemaphoreType.DMA((N_SLOT, 2)),
            pltpu.SemaphoreType.DMA((N_SLOT, 2)),
            pltpu.SemaphoreType.DMA((N_SLOT, 2)),
            pltpu.SemaphoreType.REGULAR((2,)),
            pltpu.SemaphoreType.REGULAR((2,)),
            pltpu.VMEM((4, 8, 128), jnp.float32),
            pltpu.VMEM((4, 8, 128), jnp.float32),
            pltpu.SemaphoreType.DMA((4,)),
            pltpu.SemaphoreType.DMA((4,)),
        ],
        compiler_params=pltpu.CompilerParams(
            collective_id=0,
            vmem_limit_bytes=100 * 1024 * 1024,
        ),
    )(x, w_mat)
